# baseline (speedup 1.0000x reference)
import functools

import jax
import jax.numpy as jnp
from jax import lax
from jax.experimental import pallas as pl
from jax.experimental.pallas import tpu as pltpu
from jax.experimental.pallas import tpu_sc as plsc

L = 16
NUM_TYPES = 64


def _scale_shift_call(x, sc, sh, sp, n, chunk, nc, ns):
    iters = chunk // L

    mesh = plsc.VectorSubcoreMesh(core_axis_name="c", subcore_axis_name="s")

    @functools.partial(
        pl.kernel,
        mesh=mesh,
        out_type=jax.ShapeDtypeStruct((1, n), jnp.float32),
        compiler_params=pltpu.CompilerParams(
            needs_layout_passes=False, use_tc_tiling_on_sc=False
        ),
        scratch_types=[
            pltpu.VMEM((chunk,), jnp.int32),
            pltpu.VMEM((chunk,), jnp.float32),
            pltpu.VMEM((chunk,), jnp.float32),
            pltpu.VMEM((NUM_TYPES,), jnp.float32),
            pltpu.VMEM((NUM_TYPES,), jnp.float32),
            pltpu.SemaphoreType.DMA,
        ],
    )
    def run(x_hbm, sc_hbm, sh_hbm, sp_hbm, out_hbm, sp_v, x_v, o_v, sc_v, sh_v, sem):
        wid = lax.axis_index("s") * nc + lax.axis_index("c")
        base = jnp.minimum(wid * chunk, n - chunk)

        cps = [
            pltpu.async_copy(sp_hbm.at[0, pl.ds(base, chunk)], sp_v, sem),
            pltpu.async_copy(x_hbm.at[0, pl.ds(base, chunk)], x_v, sem),
            pltpu.async_copy(sc_hbm.at[0, :], sc_v, sem),
            pltpu.async_copy(sh_hbm.at[0, :], sh_v, sem),
        ]
        for cp in cps:
            cp.wait()

        UNROLL = 8

        def body(i, carry):
            for u in range(UNROLL):
                sl = pl.ds((i * UNROLL + u) * L, L)
                idx = sp_v[sl]
                s = plsc.load_gather(sc_v, [idx])
                t = plsc.load_gather(sh_v, [idx])
                o_v[sl] = s * x_v[sl] + t
            return carry

        lax.fori_loop(0, iters // UNROLL, body, 0)
        for j in range(iters // UNROLL * UNROLL, iters):
            sl = pl.ds(j * L, L)
            idx = sp_v[sl]
            s = plsc.load_gather(sc_v, [idx])
            t = plsc.load_gather(sh_v, [idx])
            o_v[sl] = s * x_v[sl] + t

        pltpu.sync_copy(o_v, out_hbm.at[0, pl.ds(base, chunk)])

    return run(x, sc, sh, sp)


def kernel(in_field, scales, shifts, species_idx, ptr):
    del ptr
    n = in_field.shape[0]
    x = in_field.reshape(1, n)
    sp = species_idx.reshape(1, n).astype(jnp.int32)

    info = plsc.get_sparse_core_info()
    nc, ns = info.num_cores, info.num_subcores
    nw = nc * ns

    assert n % L == 0
    per_worker = -(-n // nw)
    chunk = -(-per_worker // L) * L
    chunk = min(chunk, n)

    out = _scale_shift_call(x, scales, shifts, sp, n, chunk, nc, ns)
    return out.reshape(n, 1)


# speedup vs baseline: 1.0075x; 1.0075x over previous
import functools

import jax
import jax.numpy as jnp
from jax import lax
from jax.experimental import pallas as pl
from jax.experimental.pallas import tpu as pltpu
from jax.experimental.pallas import tpu_sc as plsc

L = 16
NUM_TYPES = 64


def _scale_shift_call(x, sc, sh, sp, n, chunk, nc, ns):
    iters = chunk // L

    mesh = plsc.VectorSubcoreMesh(core_axis_name="c", subcore_axis_name="s")

    @functools.partial(
        pl.kernel,
        mesh=mesh,
        out_type=jax.ShapeDtypeStruct((1, n), jnp.float32),
        compiler_params=pltpu.CompilerParams(
            needs_layout_passes=False, use_tc_tiling_on_sc=False
        ),
        scratch_types=[
            pltpu.VMEM((chunk,), jnp.int32),
            pltpu.VMEM((chunk,), jnp.float32),
            pltpu.VMEM((chunk,), jnp.float32),
            pltpu.VMEM((NUM_TYPES,), jnp.float32),
            pltpu.VMEM((NUM_TYPES,), jnp.float32),
            pltpu.SemaphoreType.DMA,
        ],
    )
    def run(x_hbm, sc_hbm, sh_hbm, sp_hbm, out_hbm, sp_v, x_v, o_v, sc_v, sh_v, sem):
        wid = lax.axis_index("s") * nc + lax.axis_index("c")
        base = jnp.minimum(wid * chunk, n - chunk)

        cps = [
            pltpu.async_copy(sp_hbm.at[0, pl.ds(base, chunk)], sp_v, sem),
            pltpu.async_copy(x_hbm.at[0, pl.ds(base, chunk)], x_v, sem),
            pltpu.async_copy(sc_hbm.at[0, :], sc_v, sem),
            pltpu.async_copy(sh_hbm.at[0, :], sh_v, sem),
        ]
        for cp in cps:
            cp.wait()

        def body(i, carry):
            sl = pl.ds(i * L, L)
            idx = sp_v[sl]
            s = plsc.load_gather(sc_v, [idx])
            t = plsc.load_gather(sh_v, [idx])
            o_v[sl] = s * x_v[sl] + t
            return carry

        lax.fori_loop(0, iters, body, 0)

        pltpu.sync_copy(o_v, out_hbm.at[0, pl.ds(base, chunk)])

    return run(x, sc, sh, sp)


def kernel(in_field, scales, shifts, species_idx, ptr):
    del ptr
    n = in_field.shape[0]
    x = in_field.reshape(1, n)
    sp = species_idx.reshape(1, n).astype(jnp.int32)

    info = plsc.get_sparse_core_info()
    nc, ns = info.num_cores, info.num_subcores
    nw = nc * ns

    assert n % L == 0
    per_worker = -(-n // nw)
    chunk = -(-per_worker // L) * L
    chunk = min(chunk, n)

    out = _scale_shift_call(x, scales, shifts, sp, n, chunk, nc, ns)
    return out.reshape(n, 1)
